# baseline (speedup 1.0000x reference)
import functools

import jax
import jax.numpy as jnp
from jax.experimental import pallas as pl
from jax.experimental.pallas import tpu as pltpu

N = 50000
E = 800000
D = 64
NR = 9
B = 128
NCONV = 4
CUTOFF = 5.0

EBLK = 4000


def _edge_stage_body(bd_ref, gd_ref, gs_ref, cg_ref, cc_ref, cw_ref, bias_ref,
                     msg_ref):
    d = bd_ref[:, :] * CUTOFF + 0.5
    fc = 0.5 * (jnp.cos(jnp.pi * jnp.clip(d / CUTOFF, 0.0, 1.0)) + 1.0)
    ki = jax.lax.broadcasted_iota(jnp.int32, (1, 16), 1)
    freq = (ki + 1).astype(jnp.float32) * (jnp.pi / CUTOFF)
    mask = ki < NR
    rbf = jnp.where(mask, fc * jnp.sin(freq * d), 0.0)

    gb = jnp.dot(rbf, cg_ref[:, :], preferred_element_type=jnp.float32)
    cb = jnp.dot(rbf, cc_ref[:, :], preferred_element_type=jnp.float32)
    bw = jnp.dot(rbf, cw_ref[:, :], preferred_element_type=jnp.float32)

    g = gd_ref[:, :D] + gs_ref[:, :D] + gb + bias_ref[0, :][None, :]
    c = gd_ref[:, D:] + gs_ref[:, D:] + cb + bias_ref[1, :][None, :]
    sig_g = 1.0 / (1.0 + jnp.exp(-g))
    sig_c = 1.0 / (1.0 + jnp.exp(-c))
    msg_ref[:, :] = sig_g * (c * sig_c) * bw


@functools.partial(jax.jit, static_argnames=())
def _edge_stage(bond_dist2d, gd, gs, cg, cc, cw, bias):
    grid = (E // EBLK,)
    return pl.pallas_call(
        _edge_stage_body,
        grid=grid,
        in_specs=[
            pl.BlockSpec((EBLK, 1), lambda i: (i, 0)),
            pl.BlockSpec((EBLK, 2 * D), lambda i: (i, 0)),
            pl.BlockSpec((EBLK, 2 * D), lambda i: (i, 0)),
            pl.BlockSpec((16, D), lambda i: (0, 0)),
            pl.BlockSpec((16, D), lambda i: (0, 0)),
            pl.BlockSpec((16, D), lambda i: (0, 0)),
            pl.BlockSpec((2, D), lambda i: (0, 0)),
        ],
        out_specs=pl.BlockSpec((EBLK, D), lambda i: (i, 0)),
        out_shape=jax.ShapeDtypeStruct((E, D), jnp.float32),
    )(bond_dist2d, gd, gs, cg, cc, cw, bias)


def kernel(atomic_numbers, edge_index, bond_dist, atom_owners, atom_emb,
           bond_W, ag_W, Wg, bg, Wc, bc, Wout, W1, b1, W2, b2, W3, b3):
    src = edge_index[0]
    dst = edge_index[1]

    pad = jnp.zeros((16 - NR, D), jnp.float32)
    cg = []
    cc = []
    for i in range(NCONV):
        cg.append(jnp.concatenate([bond_W @ Wg[i][2 * D:], pad], axis=0))
        cc.append(jnp.concatenate([bond_W @ Wc[i][2 * D:], pad], axis=0))
    cw = jnp.concatenate([ag_W, pad], axis=0)

    x = atom_emb[atomic_numbers]
    bd2 = bond_dist[:, None]

    for i in range(NCONV):
        td = x @ jnp.concatenate([Wg[i][:D], Wc[i][:D]], axis=1)
        ts = x @ jnp.concatenate([Wg[i][D:2 * D], Wc[i][D:2 * D]], axis=1)
        gd = td[dst]
        gs = ts[src]
        bias = jnp.stack([bg[i], bc[i]])
        msg = _edge_stage(bd2, gd, gs, cg[i], cc[i], cw, bias)
        agg = jax.ops.segment_sum(msg, dst, num_segments=N)
        x = x + agg @ Wout[i]

    h = jax.nn.silu(x @ W1 + b1)
    h = jax.nn.silu(h @ W2 + b2)
    site_e = (h @ W3 + b3)[:, 0]
    e_sum = jax.ops.segment_sum(site_e, atom_owners, num_segments=B)
    counts = jnp.bincount(atom_owners, length=B).astype(jnp.float32)
    return e_sum / jnp.maximum(counts, 1.0)


# speedup vs baseline: 1.4622x; 1.4622x over previous
import functools

import jax
import jax.numpy as jnp
from jax import lax
from jax.experimental import pallas as pl
from jax.experimental.pallas import tpu as pltpu
from jax.experimental.pallas import tpu_sc as plsc

N = 50000
E = 800000
D = 64
NR = 9
B = 128
NCONV = 4
CUTOFF = 5.0

NC = 2
NS = 16
NW = NC * NS

GK = 40
GCH = E // (NW * GK)
SK = 80
SCH = E // (NS * SK)
NPT = 3136
NPAD = NS * NPT
ZBLK = 56
NBUF = 5
SSC = 25
NSC = SCH // SSC

EBLK = 4000
TBLK = 2000


def _mesh():
    return plsc.VectorSubcoreMesh(core_axis_name="c", subcore_axis_name="s")


def _sc_params():
    return pltpu.CompilerParams(use_tc_tiling_on_sc=False)


def _sc_gather(td, ts, dsti, srci):
    @functools.partial(
        pl.kernel,
        out_type=(jax.ShapeDtypeStruct((E, 2 * D), jnp.float32),
                  jax.ShapeDtypeStruct((E, 2 * D), jnp.float32)),
        mesh=_mesh(),
        compiler_params=_sc_params(),
        scratch_types=[
            pltpu.VMEM((GCH, GK), jnp.int32),
            pltpu.VMEM((NBUF, GK, 2 * D), jnp.float32),
            pltpu.SemaphoreType.DMA,
            pltpu.SemaphoreType.DMA,
        ],
    )
    def k(td_h, ts_h, di_h, si_h, gd_h, gs_h, idx_v, rows_v, gsem, wsem):
        wid = lax.axis_index("s") * NC + lax.axis_index("c")
        crow0 = wid * GCH

        for tab_h, ih, oh in ((td_h, di_h, gd_h), (ts_h, si_h, gs_h)):
            pltpu.sync_copy(ih.at[wid], idx_v)

            def gath(j, b):
                pltpu.async_copy(tab_h.at[idx_v.at[j]], rows_v.at[b], gsem)

            def wait_g(b):
                pltpu.make_async_copy(
                    tab_h.at[idx_v.at[0]], rows_v.at[b], gsem).wait()

            def wb(j, b):
                pltpu.async_copy(
                    rows_v.at[b], oh.at[pl.ds((crow0 + j) * GK, GK)], wsem)

            def wait_w(b):
                pltpu.make_async_copy(
                    rows_v.at[b], oh.at[pl.ds(crow0 * GK, GK)], wsem).wait()

            for b in range(NBUF):
                gath(b, b)

            def body(g, _):
                for b in range(NBUF):
                    wait_g(b)
                    wb(g * NBUF + b, b)
                for b in range(NBUF):
                    wait_w(b)
                    gath((g + 1) * NBUF + b, b)
                return _

            lax.fori_loop(0, GCH // NBUF - 1, body, 0)
            for b in range(NBUF):
                wait_g(b)
                wb(GCH - NBUF + b, b)
            for b in range(NBUF):
                wait_w(b)

    return k(td, ts, dsti, srci)


def _sc_scatter(msg2, dsti):
    @functools.partial(
        pl.kernel,
        out_type=jax.ShapeDtypeStruct((NC, NS, NPT, D // 2), jnp.float32),
        mesh=_mesh(),
        compiler_params=_sc_params(),
        scratch_types=[
            pltpu.VMEM((SSC, SK), jnp.int32),
            pltpu.VMEM((NBUF, SK, D // 2), jnp.float32),
            pltpu.VMEM((ZBLK, D // 2), jnp.float32),
            pltpu.VMEM_SHARED((NPAD, D // 2), jnp.float32),
            pltpu.SemaphoreType.DMA,
        ],
    )
    def k(msg_h, di_h, out_h, idx_v, upd_v, zero_v, agg_sh, lsem):
        c = lax.axis_index("c")
        s = lax.axis_index("s")

        def zrow(i, _):
            zero_v[i, 0:16] = jnp.zeros((16,), jnp.float32)
            zero_v[i, 16:32] = jnp.zeros((16,), jnp.float32)
            return _

        lax.fori_loop(0, ZBLK, zrow, 0)

        def zcopy(r, _):
            pltpu.sync_copy(zero_v, agg_sh.at[pl.ds(s * NPT + r * ZBLK, ZBLK)])
            return _

        lax.fori_loop(0, NPT // ZBLK, zcopy, 0)
        plsc.subcore_barrier()

        def load(sc, t, b):
            pltpu.async_copy(
                msg_h.at[c, pl.ds((s * SCH + sc * SSC + t) * SK, SK)],
                upd_v.at[b], lsem)

        def wait_l(b):
            pltpu.make_async_copy(
                msg_h.at[c, pl.ds(s * SCH * SK, SK)], upd_v.at[b], lsem).wait()

        def sbody(sc, _):
            pltpu.sync_copy(di_h.at[s, sc], idx_v)
            for b in range(NBUF):
                load(sc, b, b)

            def body(g, _):
                for b in range(NBUF):
                    wait_l(b)
                    pltpu.sync_copy(upd_v.at[b],
                                    agg_sh.at[idx_v.at[g * NBUF + b]],
                                    add=True)
                    load(sc, (g + 1) * NBUF + b, b)
                return _

            lax.fori_loop(0, SSC // NBUF - 1, body, 0)
            for b in range(NBUF):
                wait_l(b)
                pltpu.sync_copy(upd_v.at[b],
                                agg_sh.at[idx_v.at[SSC - NBUF + b]], add=True)
            return _

        lax.fori_loop(0, NSC, sbody, 0)

        plsc.subcore_barrier()
        pltpu.sync_copy(agg_sh.at[pl.ds(s * NPT, NPT)], out_h.at[c, s])

    return k(msg2, dsti)


def _embed_body(an_ref, emb_ref, x_ref):
    lanes = lax.broadcasted_iota(jnp.int32, (TBLK, 128), 1)
    onehot = (an_ref[:, :] == lanes).astype(jnp.float32)
    x_ref[...] = jnp.dot(onehot, emb_ref[...],
                         preferred_element_type=jnp.float32,
                    precision=lax.Precision.HIGHEST)


def _embed(an2, emb_pad):
    return pl.pallas_call(
        _embed_body,
        grid=(N // TBLK,),
        in_specs=[
            pl.BlockSpec((TBLK, 1), lambda i: (i, 0)),
            pl.BlockSpec((128, D), lambda i: (0, 0)),
        ],
        out_specs=pl.BlockSpec((TBLK, D), lambda i: (i, 0)),
        out_shape=jax.ShapeDtypeStruct((N, D), jnp.float32),
    )(an2, emb_pad)


def _tab_first_body(x_ref, wd_ref, ws_ref, td_ref, ts_ref):
    x = x_ref[...]
    td_ref[...] = jnp.dot(x, wd_ref[...], preferred_element_type=jnp.float32,
                    precision=lax.Precision.HIGHEST)
    ts_ref[...] = jnp.dot(x, ws_ref[...], preferred_element_type=jnp.float32,
                    precision=lax.Precision.HIGHEST)


def _tab_first(x, wd, ws):
    return pl.pallas_call(
        _tab_first_body,
        grid=(N // TBLK,),
        in_specs=[
            pl.BlockSpec((TBLK, D), lambda i: (i, 0)),
            pl.BlockSpec((D, 2 * D), lambda i: (0, 0)),
            pl.BlockSpec((D, 2 * D), lambda i: (0, 0)),
        ],
        out_specs=[
            pl.BlockSpec((TBLK, 2 * D), lambda i: (i, 0)),
            pl.BlockSpec((TBLK, 2 * D), lambda i: (i, 0)),
        ],
        out_shape=[
            jax.ShapeDtypeStruct((N, 2 * D), jnp.float32),
            jax.ShapeDtypeStruct((N, 2 * D), jnp.float32),
        ],
    )(x, wd, ws)


def _tab_next_body(x_ref, agg_ref, wo_ref, wd_ref, ws_ref,
                   x2_ref, td_ref, ts_ref):
    x2 = (x_ref[...]
          + jnp.dot(agg_ref[0], wo_ref[: D // 2, :],
                    preferred_element_type=jnp.float32,
                    precision=lax.Precision.HIGHEST)
          + jnp.dot(agg_ref[1], wo_ref[D // 2:, :],
                    preferred_element_type=jnp.float32,
                    precision=lax.Precision.HIGHEST))
    x2_ref[...] = x2
    td_ref[...] = jnp.dot(x2, wd_ref[...], preferred_element_type=jnp.float32,
                    precision=lax.Precision.HIGHEST)
    ts_ref[...] = jnp.dot(x2, ws_ref[...], preferred_element_type=jnp.float32,
                    precision=lax.Precision.HIGHEST)


def _tab_next(x, agg, wo, wd, ws):
    return pl.pallas_call(
        _tab_next_body,
        grid=(N // TBLK,),
        in_specs=[
            pl.BlockSpec((TBLK, D), lambda i: (i, 0)),
            pl.BlockSpec((NC, TBLK, D // 2), lambda i: (0, i, 0)),
            pl.BlockSpec((D, D), lambda i: (0, 0)),
            pl.BlockSpec((D, 2 * D), lambda i: (0, 0)),
            pl.BlockSpec((D, 2 * D), lambda i: (0, 0)),
        ],
        out_specs=[
            pl.BlockSpec((TBLK, D), lambda i: (i, 0)),
            pl.BlockSpec((TBLK, 2 * D), lambda i: (i, 0)),
            pl.BlockSpec((TBLK, 2 * D), lambda i: (i, 0)),
        ],
        out_shape=[
            jax.ShapeDtypeStruct((N, D), jnp.float32),
            jax.ShapeDtypeStruct((N, 2 * D), jnp.float32),
            jax.ShapeDtypeStruct((N, 2 * D), jnp.float32),
        ],
    )(x, agg, wo, wd, ws)


def _edge_stage_body(bd_ref, gd_ref, gs_ref, cg_ref, cc_ref, cw_ref, bias_ref,
                     msg_ref):
    d = bd_ref[:, :] * CUTOFF + 0.5
    fc = 0.5 * (jnp.cos(jnp.pi * jnp.clip(d / CUTOFF, 0.0, 1.0)) + 1.0)
    ki = lax.broadcasted_iota(jnp.int32, (1, 16), 1)
    freq = (ki + 1).astype(jnp.float32) * (jnp.pi / CUTOFF)
    rbf = jnp.where(ki < NR, fc * jnp.sin(freq * d), 0.0)

    gb = jnp.dot(rbf, cg_ref[...], preferred_element_type=jnp.float32,
                    precision=lax.Precision.HIGHEST)
    cb = jnp.dot(rbf, cc_ref[...], preferred_element_type=jnp.float32,
                    precision=lax.Precision.HIGHEST)
    bw = jnp.dot(rbf, cw_ref[...], preferred_element_type=jnp.float32,
                    precision=lax.Precision.HIGHEST)

    g = gd_ref[:, :D] + gs_ref[:, :D] + gb + bias_ref[0, :][None, :]
    c = gd_ref[:, D:] + gs_ref[:, D:] + cb + bias_ref[1, :][None, :]
    sig_g = 1.0 / (1.0 + jnp.exp(-g))
    sig_c = 1.0 / (1.0 + jnp.exp(-c))
    msg = sig_g * (c * sig_c) * bw
    msg_ref[0] = msg[:, : D // 2]
    msg_ref[1] = msg[:, D // 2:]


def _edge_stage(bond_dist2d, gd, gs, cg, cc, cw, bias):
    return pl.pallas_call(
        _edge_stage_body,
        grid=(E // EBLK,),
        in_specs=[
            pl.BlockSpec((EBLK, 1), lambda i: (i, 0)),
            pl.BlockSpec((EBLK, 2 * D), lambda i: (i, 0)),
            pl.BlockSpec((EBLK, 2 * D), lambda i: (i, 0)),
            pl.BlockSpec((16, D), lambda i: (0, 0)),
            pl.BlockSpec((16, D), lambda i: (0, 0)),
            pl.BlockSpec((16, D), lambda i: (0, 0)),
            pl.BlockSpec((2, D), lambda i: (0, 0)),
        ],
        out_specs=pl.BlockSpec((NC, EBLK, D // 2), lambda i: (0, i, 0)),
        out_shape=jax.ShapeDtypeStruct((NC, E, D // 2), jnp.float32),
    )(bond_dist2d, gd, gs, cg, cc, cw, bias)


def _readout_body(x_ref, agg_ref, wo_ref, own_ref, w1_ref, b1_ref,
                  w2_ref, b2_ref, w3_ref, b3_ref, out_ref):
    i = pl.program_id(0)

    @pl.when(i == 0)
    def _init():
        out_ref[...] = jnp.zeros_like(out_ref)

    x2 = (x_ref[...]
          + jnp.dot(agg_ref[0], wo_ref[: D // 2, :],
                    preferred_element_type=jnp.float32,
                    precision=lax.Precision.HIGHEST)
          + jnp.dot(agg_ref[1], wo_ref[D // 2:, :],
                    preferred_element_type=jnp.float32,
                    precision=lax.Precision.HIGHEST))
    h = jnp.dot(x2, w1_ref[...], preferred_element_type=jnp.float32,
                    precision=lax.Precision.HIGHEST) \
        + b1_ref[0, :][None, :]
    h = h * (1.0 / (1.0 + jnp.exp(-h)))
    h = jnp.dot(h, w2_ref[...], preferred_element_type=jnp.float32,
                    precision=lax.Precision.HIGHEST) \
        + b2_ref[0, :][None, :]
    h = h * (1.0 / (1.0 + jnp.exp(-h)))
    se = jnp.dot(h, w3_ref[...], preferred_element_type=jnp.float32,
                    precision=lax.Precision.HIGHEST) \
        + b3_ref[0, :][None, :]
    lanes = lax.broadcasted_iota(jnp.int32, (TBLK, B), 1)
    onehot = (own_ref[:, :] == lanes).astype(jnp.float32)
    out_ref[0, :] += jnp.sum(onehot * se, axis=0)
    out_ref[1, :] += jnp.sum(onehot, axis=0)


def _readout(x, agg, wo, own2, w1, b1, w2, b2, w3, b3):
    return pl.pallas_call(
        _readout_body,
        grid=(N // TBLK,),
        in_specs=[
            pl.BlockSpec((TBLK, D), lambda i: (i, 0)),
            pl.BlockSpec((NC, TBLK, D // 2), lambda i: (0, i, 0)),
            pl.BlockSpec((D, D), lambda i: (0, 0)),
            pl.BlockSpec((TBLK, 1), lambda i: (i, 0)),
            pl.BlockSpec((D, D), lambda i: (0, 0)),
            pl.BlockSpec((1, D), lambda i: (0, 0)),
            pl.BlockSpec((D, D), lambda i: (0, 0)),
            pl.BlockSpec((1, D), lambda i: (0, 0)),
            pl.BlockSpec((D, 1), lambda i: (0, 0)),
            pl.BlockSpec((1, 1), lambda i: (0, 0)),
        ],
        out_specs=pl.BlockSpec((2, B), lambda i: (0, 0)),
        out_shape=jax.ShapeDtypeStruct((2, B), jnp.float32),
    )(x, agg, wo, own2, w1, b1, w2, b2, w3, b3)


def kernel(atomic_numbers, edge_index, bond_dist, atom_owners, atom_emb,
           bond_W, ag_W, Wg, bg, Wc, bc, Wout, W1, b1, W2, b2, W3, b3):
    src = edge_index[0].astype(jnp.int32)
    dst = edge_index[1].astype(jnp.int32)
    dst_g = dst.reshape(NW, GCH, GK)
    src_g = src.reshape(NW, GCH, GK)
    dst_s = dst.reshape(NS, NSC, SSC, SK)

    pad = jnp.zeros((16 - NR, D), jnp.float32)
    cg, cc, wd, ws = [], [], [], []
    for i in range(NCONV):
        cg.append(jnp.concatenate([bond_W @ Wg[i][2 * D:], pad], axis=0))
        cc.append(jnp.concatenate([bond_W @ Wc[i][2 * D:], pad], axis=0))
        wd.append(jnp.concatenate([Wg[i][:D], Wc[i][:D]], axis=1))
        ws.append(jnp.concatenate([Wg[i][D:2 * D], Wc[i][D:2 * D]], axis=1))
    cw = jnp.concatenate([ag_W, pad], axis=0)
    emb_pad = jnp.concatenate(
        [atom_emb, jnp.zeros((128 - atom_emb.shape[0], D), jnp.float32)], 0)

    x = _embed(atomic_numbers.astype(jnp.int32)[:, None], emb_pad)
    bd2 = bond_dist[:, None]

    agg = None
    for i in range(NCONV):
        if i == 0:
            td, ts = _tab_first(x, wd[i], ws[i])
        else:
            x, td, ts = _tab_next(x, agg, Wout[i - 1], wd[i], ws[i])
        gd, gs = _sc_gather(td, ts, dst_g, src_g)
        bias = jnp.stack([bg[i], bc[i]])
        msg2 = _edge_stage(bd2, gd, gs, cg[i], cc[i], cw, bias)
        agg = _sc_scatter(msg2, dst_s).reshape(NC, NPAD, D // 2)

    out = _readout(x, agg, Wout[NCONV - 1],
                   atom_owners.astype(jnp.int32)[:, None],
                   W1, b1[None, :], W2, b2[None, :], W3, b3[None, :])
    return out[0] / jnp.maximum(out[1], 1.0)


# speedup vs baseline: 2.6832x; 1.8350x over previous
import functools

import jax
import jax.numpy as jnp
from jax import lax
from jax.experimental import pallas as pl
from jax.experimental.pallas import tpu as pltpu
from jax.experimental.pallas import tpu_sc as plsc

N = 50000
E = 800000
D = 64
NR = 9
B = 128
NCONV = 4
CUTOFF = 5.0

NC = 2
NS = 16
NW = NC * NS

GK = 40
GCH = E // (NW * GK)
SK = 80
SCH = E // (NS * SK)
NPT = 3136
NPAD = NS * NPT
ZBLK = 56
NBUF = 5
SSC = 25
NSC = SCH // SSC

EBLK = 4000
TBLK = 2000


def _mesh():
    return plsc.VectorSubcoreMesh(core_axis_name="c", subcore_axis_name="s")


def _sc_params():
    return pltpu.CompilerParams(use_tc_tiling_on_sc=False)


def _sc_gather(td, ts, dsti, srci):
    @functools.partial(
        pl.kernel,
        out_type=(jax.ShapeDtypeStruct((E, 2 * D), jnp.float32),
                  jax.ShapeDtypeStruct((E, 2 * D), jnp.float32)),
        mesh=_mesh(),
        compiler_params=_sc_params(),
        scratch_types=[
            pltpu.VMEM((GCH, GK), jnp.int32),
            pltpu.VMEM((NBUF, GK, 2 * D), jnp.float32),
            pltpu.SemaphoreType.DMA,
            pltpu.SemaphoreType.DMA,
        ],
    )
    def k(td_h, ts_h, di_h, si_h, gd_h, gs_h, idx_v, rows_v, gsem, wsem):
        wid = lax.axis_index("s") * NC + lax.axis_index("c")
        crow0 = wid * GCH

        for tab_h, ih, oh in ((td_h, di_h, gd_h), (ts_h, si_h, gs_h)):
            pltpu.sync_copy(ih.at[wid], idx_v)

            def gath(j, b):
                pltpu.async_copy(tab_h.at[idx_v.at[j]], rows_v.at[b], gsem)

            def wait_g(b):
                pltpu.make_async_copy(
                    tab_h.at[idx_v.at[0]], rows_v.at[b], gsem).wait()

            def wb(j, b):
                pltpu.async_copy(
                    rows_v.at[b], oh.at[pl.ds((crow0 + j) * GK, GK)], wsem)

            def wait_w(b):
                pltpu.make_async_copy(
                    rows_v.at[b], oh.at[pl.ds(crow0 * GK, GK)], wsem).wait()

            for b in range(NBUF):
                gath(b, b)

            def body(g, _):
                for b in range(NBUF):
                    wait_g(b)
                    wb(g * NBUF + b, b)
                for b in range(NBUF):
                    wait_w(b)
                    gath((g + 1) * NBUF + b, b)
                return _

            lax.fori_loop(0, GCH // NBUF - 1, body, 0)
            for b in range(NBUF):
                wait_g(b)
                wb(GCH - NBUF + b, b)
            for b in range(NBUF):
                wait_w(b)

    return k(td, ts, dsti, srci)


def _sc_scatter(msg2, dsti):
    @functools.partial(
        pl.kernel,
        out_type=jax.ShapeDtypeStruct((NC, NS, NPT, D // 2), jnp.float32),
        mesh=_mesh(),
        compiler_params=_sc_params(),
        scratch_types=[
            pltpu.VMEM((SSC, SK), jnp.int32),
            pltpu.VMEM((NBUF, SK, D // 2), jnp.float32),
            pltpu.VMEM((ZBLK, D // 2), jnp.float32),
            pltpu.VMEM_SHARED((NPAD, D // 2), jnp.float32),
            pltpu.SemaphoreType.DMA,
        ],
    )
    def k(msg_h, di_h, out_h, idx_v, upd_v, zero_v, agg_sh, lsem):
        c = lax.axis_index("c")
        s = lax.axis_index("s")

        def zrow(i, _):
            zero_v[i, 0:16] = jnp.zeros((16,), jnp.float32)
            zero_v[i, 16:32] = jnp.zeros((16,), jnp.float32)
            return _

        lax.fori_loop(0, ZBLK, zrow, 0)

        def zcopy(r, _):
            pltpu.sync_copy(zero_v, agg_sh.at[pl.ds(s * NPT + r * ZBLK, ZBLK)])
            return _

        lax.fori_loop(0, NPT // ZBLK, zcopy, 0)
        plsc.subcore_barrier()

        def load(sc, t, b):
            pltpu.async_copy(
                msg_h.at[c, pl.ds((s * SCH + sc * SSC + t) * SK, SK)],
                upd_v.at[b], lsem)

        def wait_l(b):
            pltpu.make_async_copy(
                msg_h.at[c, pl.ds(s * SCH * SK, SK)], upd_v.at[b], lsem).wait()

        def sbody(sc, _):
            pltpu.sync_copy(di_h.at[s, sc], idx_v)
            for b in range(NBUF):
                load(sc, b, b)

            def body(g, _):
                for b in range(NBUF):
                    wait_l(b)
                    pltpu.sync_copy(upd_v.at[b],
                                    agg_sh.at[idx_v.at[g * NBUF + b]],
                                    add=True)
                    load(sc, (g + 1) * NBUF + b, b)
                return _

            lax.fori_loop(0, SSC // NBUF - 1, body, 0)
            for b in range(NBUF):
                wait_l(b)
                pltpu.sync_copy(upd_v.at[b],
                                agg_sh.at[idx_v.at[SSC - NBUF + b]], add=True)
            return _

        lax.fori_loop(0, NSC, sbody, 0)

        plsc.subcore_barrier()
        pltpu.sync_copy(agg_sh.at[pl.ds(s * NPT, NPT)], out_h.at[c, s])

    return k(msg2, dsti)


def _embed_body(an_ref, emb_ref, x_ref):
    lanes = lax.broadcasted_iota(jnp.int32, (TBLK, 128), 1)
    onehot = (an_ref[:, :] == lanes).astype(jnp.float32)
    x_ref[...] = jnp.dot(onehot, emb_ref[...],
                         preferred_element_type=jnp.float32,
                    precision=lax.Precision.HIGHEST)


def _embed(an2, emb_pad):
    return pl.pallas_call(
        _embed_body,
        grid=(N // TBLK,),
        in_specs=[
            pl.BlockSpec((TBLK, 1), lambda i: (i, 0)),
            pl.BlockSpec((128, D), lambda i: (0, 0)),
        ],
        out_specs=pl.BlockSpec((TBLK, D), lambda i: (i, 0)),
        out_shape=jax.ShapeDtypeStruct((N, D), jnp.float32),
    )(an2, emb_pad)


def _tab_first_body(x_ref, wd_ref, ws_ref, td_ref, ts_ref):
    x = x_ref[...]
    td_ref[...] = jnp.dot(x, wd_ref[...], preferred_element_type=jnp.float32,
                    precision=lax.Precision.HIGHEST)
    ts_ref[...] = jnp.dot(x, ws_ref[...], preferred_element_type=jnp.float32,
                    precision=lax.Precision.HIGHEST)


def _tab_first(x, wd, ws):
    return pl.pallas_call(
        _tab_first_body,
        grid=(N // TBLK,),
        in_specs=[
            pl.BlockSpec((TBLK, D), lambda i: (i, 0)),
            pl.BlockSpec((D, 2 * D), lambda i: (0, 0)),
            pl.BlockSpec((D, 2 * D), lambda i: (0, 0)),
        ],
        out_specs=[
            pl.BlockSpec((TBLK, 2 * D), lambda i: (i, 0)),
            pl.BlockSpec((TBLK, 2 * D), lambda i: (i, 0)),
        ],
        out_shape=[
            jax.ShapeDtypeStruct((N, 2 * D), jnp.float32),
            jax.ShapeDtypeStruct((N, 2 * D), jnp.float32),
        ],
    )(x, wd, ws)


def _tab_next_body(x_ref, agg_ref, wo_ref, wd_ref, ws_ref,
                   x2_ref, td_ref, ts_ref):
    x2 = (x_ref[...]
          + jnp.dot(agg_ref[0], wo_ref[: D // 2, :],
                    preferred_element_type=jnp.float32,
                    precision=lax.Precision.HIGHEST)
          + jnp.dot(agg_ref[1], wo_ref[D // 2:, :],
                    preferred_element_type=jnp.float32,
                    precision=lax.Precision.HIGHEST))
    x2_ref[...] = x2
    td_ref[...] = jnp.dot(x2, wd_ref[...], preferred_element_type=jnp.float32,
                    precision=lax.Precision.HIGHEST)
    ts_ref[...] = jnp.dot(x2, ws_ref[...], preferred_element_type=jnp.float32,
                    precision=lax.Precision.HIGHEST)


def _tab_next(x, agg, wo, wd, ws):
    return pl.pallas_call(
        _tab_next_body,
        grid=(N // TBLK,),
        in_specs=[
            pl.BlockSpec((TBLK, D), lambda i: (i, 0)),
            pl.BlockSpec((NC, TBLK, D // 2), lambda i: (0, i, 0)),
            pl.BlockSpec((D, D), lambda i: (0, 0)),
            pl.BlockSpec((D, 2 * D), lambda i: (0, 0)),
            pl.BlockSpec((D, 2 * D), lambda i: (0, 0)),
        ],
        out_specs=[
            pl.BlockSpec((TBLK, D), lambda i: (i, 0)),
            pl.BlockSpec((TBLK, 2 * D), lambda i: (i, 0)),
            pl.BlockSpec((TBLK, 2 * D), lambda i: (i, 0)),
        ],
        out_shape=[
            jax.ShapeDtypeStruct((N, D), jnp.float32),
            jax.ShapeDtypeStruct((N, 2 * D), jnp.float32),
            jax.ShapeDtypeStruct((N, 2 * D), jnp.float32),
        ],
    )(x, agg, wo, wd, ws)


def _sinpi(t):
    n = jnp.floor(t + 0.5)
    x = (t - n) * jnp.float32(jnp.pi)
    y = x * x
    s = x * (1.0 + y * (-1.0 / 6.0 + y * (1.0 / 120.0 + y * (
        -1.0 / 5040.0 + y * (1.0 / 362880.0 - y * (1.0 / 39916800.0))))))
    h = n * 0.5
    odd = (h - jnp.floor(h)) > 0.25
    return jnp.where(odd, -s, s)


def _edge_stage_body(bd_ref, gd_ref, gs_ref, call_ref, bias_ref, msg_ref):
    d = bd_ref[:, :] * CUTOFF + 0.5
    u = d * (1.0 / CUTOFF)
    fc = 0.5 * (_sinpi(0.5 - jnp.minimum(u, 1.0)) + 1.0)
    ki = lax.broadcasted_iota(jnp.int32, (1, 16), 1)
    t = (ki + 1).astype(jnp.float32) * u
    rbf = jnp.where(ki < NR, fc * _sinpi(t), 0.0)

    bond = jnp.dot(rbf, call_ref[...], preferred_element_type=jnp.float32,
                   precision=lax.Precision.HIGHEST)

    g = gd_ref[:, :D] + gs_ref[:, :D] + bond[:, :D] + bias_ref[0, :][None, :]
    c = gd_ref[:, D:] + gs_ref[:, D:] + bond[:, D:2 * D] \
        + bias_ref[1, :][None, :]
    bw = bond[:, 2 * D:]
    sig_g = 1.0 / (1.0 + jnp.exp(-g))
    sig_c = 1.0 / (1.0 + jnp.exp(-c))
    msg = sig_g * (c * sig_c) * bw
    msg_ref[0] = msg[:, : D // 2]
    msg_ref[1] = msg[:, D // 2:]


def _edge_stage(bond_dist2d, gd, gs, call, bias):
    return pl.pallas_call(
        _edge_stage_body,
        grid=(E // EBLK,),
        in_specs=[
            pl.BlockSpec((EBLK, 1), lambda i: (i, 0)),
            pl.BlockSpec((EBLK, 2 * D), lambda i: (i, 0)),
            pl.BlockSpec((EBLK, 2 * D), lambda i: (i, 0)),
            pl.BlockSpec((16, 3 * D), lambda i: (0, 0)),
            pl.BlockSpec((2, D), lambda i: (0, 0)),
        ],
        out_specs=pl.BlockSpec((NC, EBLK, D // 2), lambda i: (0, i, 0)),
        out_shape=jax.ShapeDtypeStruct((NC, E, D // 2), jnp.float32),
    )(bond_dist2d, gd, gs, call, bias)


def _readout_body(x_ref, agg_ref, wo_ref, own_ref, w1_ref, b1_ref,
                  w2_ref, b2_ref, w3_ref, b3_ref, out_ref):
    i = pl.program_id(0)

    @pl.when(i == 0)
    def _init():
        out_ref[...] = jnp.zeros_like(out_ref)

    x2 = (x_ref[...]
          + jnp.dot(agg_ref[0], wo_ref[: D // 2, :],
                    preferred_element_type=jnp.float32,
                    precision=lax.Precision.HIGHEST)
          + jnp.dot(agg_ref[1], wo_ref[D // 2:, :],
                    preferred_element_type=jnp.float32,
                    precision=lax.Precision.HIGHEST))
    h = jnp.dot(x2, w1_ref[...], preferred_element_type=jnp.float32,
                    precision=lax.Precision.HIGHEST) \
        + b1_ref[0, :][None, :]
    h = h * (1.0 / (1.0 + jnp.exp(-h)))
    h = jnp.dot(h, w2_ref[...], preferred_element_type=jnp.float32,
                    precision=lax.Precision.HIGHEST) \
        + b2_ref[0, :][None, :]
    h = h * (1.0 / (1.0 + jnp.exp(-h)))
    se = jnp.dot(h, w3_ref[...], preferred_element_type=jnp.float32,
                    precision=lax.Precision.HIGHEST) \
        + b3_ref[0, :][None, :]
    lanes = lax.broadcasted_iota(jnp.int32, (TBLK, B), 1)
    onehot = (own_ref[:, :] == lanes).astype(jnp.float32)
    out_ref[0, :] += jnp.sum(onehot * se, axis=0)
    out_ref[1, :] += jnp.sum(onehot, axis=0)


def _readout(x, agg, wo, own2, w1, b1, w2, b2, w3, b3):
    return pl.pallas_call(
        _readout_body,
        grid=(N // TBLK,),
        in_specs=[
            pl.BlockSpec((TBLK, D), lambda i: (i, 0)),
            pl.BlockSpec((NC, TBLK, D // 2), lambda i: (0, i, 0)),
            pl.BlockSpec((D, D), lambda i: (0, 0)),
            pl.BlockSpec((TBLK, 1), lambda i: (i, 0)),
            pl.BlockSpec((D, D), lambda i: (0, 0)),
            pl.BlockSpec((1, D), lambda i: (0, 0)),
            pl.BlockSpec((D, D), lambda i: (0, 0)),
            pl.BlockSpec((1, D), lambda i: (0, 0)),
            pl.BlockSpec((D, 1), lambda i: (0, 0)),
            pl.BlockSpec((1, 1), lambda i: (0, 0)),
        ],
        out_specs=pl.BlockSpec((2, B), lambda i: (0, 0)),
        out_shape=jax.ShapeDtypeStruct((2, B), jnp.float32),
    )(x, agg, wo, own2, w1, b1, w2, b2, w3, b3)


def kernel(atomic_numbers, edge_index, bond_dist, atom_owners, atom_emb,
           bond_W, ag_W, Wg, bg, Wc, bc, Wout, W1, b1, W2, b2, W3, b3):
    src = edge_index[0].astype(jnp.int32)
    dst = edge_index[1].astype(jnp.int32)
    dst_g = dst.reshape(NW, GCH, GK)
    src_g = src.reshape(NW, GCH, GK)
    dst_s = dst.reshape(NS, NSC, SSC, SK)

    pad = jnp.zeros((16 - NR, D), jnp.float32)
    cw = jnp.concatenate([ag_W, pad], axis=0)
    call, wd, ws = [], [], []
    for i in range(NCONV):
        cg = jnp.concatenate([bond_W @ Wg[i][2 * D:], pad], axis=0)
        cc = jnp.concatenate([bond_W @ Wc[i][2 * D:], pad], axis=0)
        call.append(jnp.concatenate([cg, cc, cw], axis=1))
        wd.append(jnp.concatenate([Wg[i][:D], Wc[i][:D]], axis=1))
        ws.append(jnp.concatenate([Wg[i][D:2 * D], Wc[i][D:2 * D]], axis=1))
    emb_pad = jnp.concatenate(
        [atom_emb, jnp.zeros((128 - atom_emb.shape[0], D), jnp.float32)], 0)

    x = _embed(atomic_numbers.astype(jnp.int32)[:, None], emb_pad)
    bd2 = bond_dist[:, None]

    agg = None
    for i in range(NCONV):
        if i == 0:
            td, ts = _tab_first(x, wd[i], ws[i])
        else:
            x, td, ts = _tab_next(x, agg, Wout[i - 1], wd[i], ws[i])
        gd, gs = _sc_gather(td, ts, dst_g, src_g)
        bias = jnp.stack([bg[i], bc[i]])
        msg2 = _edge_stage(bd2, gd, gs, call[i], bias)
        agg = _sc_scatter(msg2, dst_s).reshape(NC, NPAD, D // 2)

    out = _readout(x, agg, Wout[NCONV - 1],
                   atom_owners.astype(jnp.int32)[:, None],
                   W1, b1[None, :], W2, b2[None, :], W3, b3[None, :])
    return out[0] / jnp.maximum(out[1], 1.0)


# speedup vs baseline: 2.7020x; 1.0070x over previous
import functools

import jax
import jax.numpy as jnp
from jax import lax
from jax.experimental import pallas as pl
from jax.experimental.pallas import tpu as pltpu
from jax.experimental.pallas import tpu_sc as plsc

N = 50000
E = 800000
D = 64
NR = 9
B = 128
NCONV = 4
CUTOFF = 5.0

NC = 2
NS = 16
NW = NC * NS

GK = 40
GCH = E // (NW * GK)
GSC = 25
GNS = GCH // GSC
SK = 80
SCH = E // (NS * SK)
NPT = 3136
NPAD = NS * NPT
ZBLK = 56
NBUF = 5
SSC = 25
NSC = SCH // SSC

EBLK = 4000
TBLK = 2000


def _mesh():
    return plsc.VectorSubcoreMesh(core_axis_name="c", subcore_axis_name="s")


def _sc_params():
    return pltpu.CompilerParams(use_tc_tiling_on_sc=False)


def _sc_gather(td, ts, dsti, srci):
    @functools.partial(
        pl.kernel,
        out_type=(jax.ShapeDtypeStruct((E, 2 * D), jnp.float32),
                  jax.ShapeDtypeStruct((E, 2 * D), jnp.float32)),
        mesh=_mesh(),
        compiler_params=_sc_params(),
        scratch_types=[
            pltpu.VMEM((GSC, GK), jnp.int32),
            pltpu.VMEM((GSC, GK, 2 * D), jnp.float32),
            pltpu.SemaphoreType.DMA,
            pltpu.SemaphoreType.DMA,
        ],
    )
    def k(td_h, ts_h, di_h, si_h, gd_h, gs_h, idx_v, rows_v, gsem, wsem):
        wid = lax.axis_index("s") * NC + lax.axis_index("c")
        crow0 = wid * GCH

        for tab_h, ih, oh in ((td_h, di_h, gd_h), (ts_h, si_h, gs_h)):

            def sbody(sc, _):
                pltpu.sync_copy(ih.at[wid, sc], idx_v)
                for b in range(GSC):
                    pltpu.async_copy(tab_h.at[idx_v.at[b]], rows_v.at[b],
                                     gsem)
                for b in range(GSC):
                    pltpu.make_async_copy(
                        tab_h.at[idx_v.at[b]], rows_v.at[b], gsem).wait()
                    pltpu.async_copy(
                        rows_v.at[b],
                        oh.at[pl.ds((crow0 + sc * GSC + b) * GK, GK)], wsem)
                for b in range(GSC):
                    pltpu.make_async_copy(
                        rows_v.at[b], oh.at[pl.ds(crow0 * GK, GK)],
                        wsem).wait()
                return _

            lax.fori_loop(0, GNS, sbody, 0)

    return k(td, ts, dsti, srci)


def _sc_scatter(msg2, dsti):
    @functools.partial(
        pl.kernel,
        out_type=jax.ShapeDtypeStruct((NC, NS, NPT, D // 2), jnp.float32),
        mesh=_mesh(),
        compiler_params=_sc_params(),
        scratch_types=[
            pltpu.VMEM((SSC, SK), jnp.int32),
            pltpu.VMEM((NBUF, SK, D // 2), jnp.float32),
            pltpu.VMEM((ZBLK, D // 2), jnp.float32),
            pltpu.VMEM_SHARED((NPAD, D // 2), jnp.float32),
            pltpu.SemaphoreType.DMA,
        ],
    )
    def k(msg_h, di_h, out_h, idx_v, upd_v, zero_v, agg_sh, lsem):
        c = lax.axis_index("c")
        s = lax.axis_index("s")

        def zrow(i, _):
            zero_v[i, 0:16] = jnp.zeros((16,), jnp.float32)
            zero_v[i, 16:32] = jnp.zeros((16,), jnp.float32)
            return _

        lax.fori_loop(0, ZBLK, zrow, 0)

        def zcopy(r, _):
            pltpu.sync_copy(zero_v, agg_sh.at[pl.ds(s * NPT + r * ZBLK, ZBLK)])
            return _

        lax.fori_loop(0, NPT // ZBLK, zcopy, 0)
        plsc.subcore_barrier()

        def load(sc, t, b):
            pltpu.async_copy(
                msg_h.at[c, pl.ds((s * SCH + sc * SSC + t) * SK, SK)],
                upd_v.at[b], lsem)

        def wait_l(b):
            pltpu.make_async_copy(
                msg_h.at[c, pl.ds(s * SCH * SK, SK)], upd_v.at[b], lsem).wait()

        def sbody(sc, _):
            pltpu.sync_copy(di_h.at[s, sc], idx_v)
            for b in range(NBUF):
                load(sc, b, b)

            def body(g, _):
                for b in range(NBUF):
                    wait_l(b)
                    pltpu.sync_copy(upd_v.at[b],
                                    agg_sh.at[idx_v.at[g * NBUF + b]],
                                    add=True)
                    load(sc, (g + 1) * NBUF + b, b)
                return _

            lax.fori_loop(0, SSC // NBUF - 1, body, 0)
            for b in range(NBUF):
                wait_l(b)
                pltpu.sync_copy(upd_v.at[b],
                                agg_sh.at[idx_v.at[SSC - NBUF + b]], add=True)
            return _

        lax.fori_loop(0, NSC, sbody, 0)

        plsc.subcore_barrier()
        pltpu.sync_copy(agg_sh.at[pl.ds(s * NPT, NPT)], out_h.at[c, s])

    return k(msg2, dsti)


def _embed_body(an_ref, emb_ref, x_ref):
    lanes = lax.broadcasted_iota(jnp.int32, (TBLK, 128), 1)
    onehot = (an_ref[:, :] == lanes).astype(jnp.float32)
    x_ref[...] = jnp.dot(onehot, emb_ref[...],
                         preferred_element_type=jnp.float32,
                    precision=lax.Precision.HIGHEST)


def _embed(an2, emb_pad):
    return pl.pallas_call(
        _embed_body,
        grid=(N // TBLK,),
        in_specs=[
            pl.BlockSpec((TBLK, 1), lambda i: (i, 0)),
            pl.BlockSpec((128, D), lambda i: (0, 0)),
        ],
        out_specs=pl.BlockSpec((TBLK, D), lambda i: (i, 0)),
        out_shape=jax.ShapeDtypeStruct((N, D), jnp.float32),
    )(an2, emb_pad)


def _tab_first_body(x_ref, wd_ref, ws_ref, td_ref, ts_ref):
    x = x_ref[...]
    td_ref[...] = jnp.dot(x, wd_ref[...], preferred_element_type=jnp.float32,
                    precision=lax.Precision.HIGHEST)
    ts_ref[...] = jnp.dot(x, ws_ref[...], preferred_element_type=jnp.float32,
                    precision=lax.Precision.HIGHEST)


def _tab_first(x, wd, ws):
    return pl.pallas_call(
        _tab_first_body,
        grid=(N // TBLK,),
        in_specs=[
            pl.BlockSpec((TBLK, D), lambda i: (i, 0)),
            pl.BlockSpec((D, 2 * D), lambda i: (0, 0)),
            pl.BlockSpec((D, 2 * D), lambda i: (0, 0)),
        ],
        out_specs=[
            pl.BlockSpec((TBLK, 2 * D), lambda i: (i, 0)),
            pl.BlockSpec((TBLK, 2 * D), lambda i: (i, 0)),
        ],
        out_shape=[
            jax.ShapeDtypeStruct((N, 2 * D), jnp.float32),
            jax.ShapeDtypeStruct((N, 2 * D), jnp.float32),
        ],
    )(x, wd, ws)


def _tab_next_body(x_ref, agg_ref, wo_ref, wd_ref, ws_ref,
                   x2_ref, td_ref, ts_ref):
    x2 = (x_ref[...]
          + jnp.dot(agg_ref[0], wo_ref[: D // 2, :],
                    preferred_element_type=jnp.float32,
                    precision=lax.Precision.HIGHEST)
          + jnp.dot(agg_ref[1], wo_ref[D // 2:, :],
                    preferred_element_type=jnp.float32,
                    precision=lax.Precision.HIGHEST))
    x2_ref[...] = x2
    td_ref[...] = jnp.dot(x2, wd_ref[...], preferred_element_type=jnp.float32,
                    precision=lax.Precision.HIGHEST)
    ts_ref[...] = jnp.dot(x2, ws_ref[...], preferred_element_type=jnp.float32,
                    precision=lax.Precision.HIGHEST)


def _tab_next(x, agg, wo, wd, ws):
    return pl.pallas_call(
        _tab_next_body,
        grid=(N // TBLK,),
        in_specs=[
            pl.BlockSpec((TBLK, D), lambda i: (i, 0)),
            pl.BlockSpec((NC, TBLK, D // 2), lambda i: (0, i, 0)),
            pl.BlockSpec((D, D), lambda i: (0, 0)),
            pl.BlockSpec((D, 2 * D), lambda i: (0, 0)),
            pl.BlockSpec((D, 2 * D), lambda i: (0, 0)),
        ],
        out_specs=[
            pl.BlockSpec((TBLK, D), lambda i: (i, 0)),
            pl.BlockSpec((TBLK, 2 * D), lambda i: (i, 0)),
            pl.BlockSpec((TBLK, 2 * D), lambda i: (i, 0)),
        ],
        out_shape=[
            jax.ShapeDtypeStruct((N, D), jnp.float32),
            jax.ShapeDtypeStruct((N, 2 * D), jnp.float32),
            jax.ShapeDtypeStruct((N, 2 * D), jnp.float32),
        ],
    )(x, agg, wo, wd, ws)


def _sinpi(t):
    n = jnp.floor(t + 0.5)
    x = (t - n) * jnp.float32(jnp.pi)
    y = x * x
    s = x * (1.0 + y * (-1.0 / 6.0 + y * (1.0 / 120.0 + y * (
        -1.0 / 5040.0 + y * (1.0 / 362880.0 - y * (1.0 / 39916800.0))))))
    h = n * 0.5
    odd = (h - jnp.floor(h)) > 0.25
    return jnp.where(odd, -s, s)


def _edge_stage_body(bd_ref, gd_ref, gs_ref, call_ref, bias_ref, msg_ref):
    d = bd_ref[:, :] * CUTOFF + 0.5
    u = d * (1.0 / CUTOFF)
    fc = 0.5 * (_sinpi(0.5 - jnp.minimum(u, 1.0)) + 1.0)
    ki = lax.broadcasted_iota(jnp.int32, (1, 16), 1)
    t = (ki + 1).astype(jnp.float32) * u
    rbf = jnp.where(ki < NR, fc * _sinpi(t), 0.0)

    bond = jnp.dot(rbf, call_ref[...], preferred_element_type=jnp.float32,
                   precision=lax.Precision.HIGHEST)

    g = gd_ref[:, :D] + gs_ref[:, :D] + bond[:, :D] + bias_ref[0, :][None, :]
    c = gd_ref[:, D:] + gs_ref[:, D:] + bond[:, D:2 * D] \
        + bias_ref[1, :][None, :]
    bw = bond[:, 2 * D:]
    sig_g = 1.0 / (1.0 + jnp.exp(-g))
    sig_c = 1.0 / (1.0 + jnp.exp(-c))
    msg = sig_g * (c * sig_c) * bw
    msg_ref[0] = msg[:, : D // 2]
    msg_ref[1] = msg[:, D // 2:]


def _edge_stage(bond_dist2d, gd, gs, call, bias):
    return pl.pallas_call(
        _edge_stage_body,
        grid=(E // EBLK,),
        in_specs=[
            pl.BlockSpec((EBLK, 1), lambda i: (i, 0)),
            pl.BlockSpec((EBLK, 2 * D), lambda i: (i, 0)),
            pl.BlockSpec((EBLK, 2 * D), lambda i: (i, 0)),
            pl.BlockSpec((16, 3 * D), lambda i: (0, 0)),
            pl.BlockSpec((2, D), lambda i: (0, 0)),
        ],
        out_specs=pl.BlockSpec((NC, EBLK, D // 2), lambda i: (0, i, 0)),
        out_shape=jax.ShapeDtypeStruct((NC, E, D // 2), jnp.float32),
    )(bond_dist2d, gd, gs, call, bias)


def _readout_body(x_ref, agg_ref, wo_ref, own_ref, w1_ref, b1_ref,
                  w2_ref, b2_ref, w3_ref, b3_ref, out_ref):
    i = pl.program_id(0)

    @pl.when(i == 0)
    def _init():
        out_ref[...] = jnp.zeros_like(out_ref)

    x2 = (x_ref[...]
          + jnp.dot(agg_ref[0], wo_ref[: D // 2, :],
                    preferred_element_type=jnp.float32,
                    precision=lax.Precision.HIGHEST)
          + jnp.dot(agg_ref[1], wo_ref[D // 2:, :],
                    preferred_element_type=jnp.float32,
                    precision=lax.Precision.HIGHEST))
    h = jnp.dot(x2, w1_ref[...], preferred_element_type=jnp.float32,
                    precision=lax.Precision.HIGHEST) \
        + b1_ref[0, :][None, :]
    h = h * (1.0 / (1.0 + jnp.exp(-h)))
    h = jnp.dot(h, w2_ref[...], preferred_element_type=jnp.float32,
                    precision=lax.Precision.HIGHEST) \
        + b2_ref[0, :][None, :]
    h = h * (1.0 / (1.0 + jnp.exp(-h)))
    se = jnp.dot(h, w3_ref[...], preferred_element_type=jnp.float32,
                    precision=lax.Precision.HIGHEST) \
        + b3_ref[0, :][None, :]
    lanes = lax.broadcasted_iota(jnp.int32, (TBLK, B), 1)
    onehot = (own_ref[:, :] == lanes).astype(jnp.float32)
    out_ref[0, :] += jnp.sum(onehot * se, axis=0)
    out_ref[1, :] += jnp.sum(onehot, axis=0)


def _readout(x, agg, wo, own2, w1, b1, w2, b2, w3, b3):
    return pl.pallas_call(
        _readout_body,
        grid=(N // TBLK,),
        in_specs=[
            pl.BlockSpec((TBLK, D), lambda i: (i, 0)),
            pl.BlockSpec((NC, TBLK, D // 2), lambda i: (0, i, 0)),
            pl.BlockSpec((D, D), lambda i: (0, 0)),
            pl.BlockSpec((TBLK, 1), lambda i: (i, 0)),
            pl.BlockSpec((D, D), lambda i: (0, 0)),
            pl.BlockSpec((1, D), lambda i: (0, 0)),
            pl.BlockSpec((D, D), lambda i: (0, 0)),
            pl.BlockSpec((1, D), lambda i: (0, 0)),
            pl.BlockSpec((D, 1), lambda i: (0, 0)),
            pl.BlockSpec((1, 1), lambda i: (0, 0)),
        ],
        out_specs=pl.BlockSpec((2, B), lambda i: (0, 0)),
        out_shape=jax.ShapeDtypeStruct((2, B), jnp.float32),
    )(x, agg, wo, own2, w1, b1, w2, b2, w3, b3)


def kernel(atomic_numbers, edge_index, bond_dist, atom_owners, atom_emb,
           bond_W, ag_W, Wg, bg, Wc, bc, Wout, W1, b1, W2, b2, W3, b3):
    src = edge_index[0].astype(jnp.int32)
    dst = edge_index[1].astype(jnp.int32)
    dst_g = dst.reshape(NW, GNS, GSC, GK)
    src_g = src.reshape(NW, GNS, GSC, GK)
    dst_s = dst.reshape(NS, NSC, SSC, SK)

    pad = jnp.zeros((16 - NR, D), jnp.float32)
    cw = jnp.concatenate([ag_W, pad], axis=0)
    call, wd, ws = [], [], []
    for i in range(NCONV):
        cg = jnp.concatenate([bond_W @ Wg[i][2 * D:], pad], axis=0)
        cc = jnp.concatenate([bond_W @ Wc[i][2 * D:], pad], axis=0)
        call.append(jnp.concatenate([cg, cc, cw], axis=1))
        wd.append(jnp.concatenate([Wg[i][:D], Wc[i][:D]], axis=1))
        ws.append(jnp.concatenate([Wg[i][D:2 * D], Wc[i][D:2 * D]], axis=1))
    emb_pad = jnp.concatenate(
        [atom_emb, jnp.zeros((128 - atom_emb.shape[0], D), jnp.float32)], 0)

    x = _embed(atomic_numbers.astype(jnp.int32)[:, None], emb_pad)
    bd2 = bond_dist[:, None]

    agg = None
    for i in range(NCONV):
        if i == 0:
            td, ts = _tab_first(x, wd[i], ws[i])
        else:
            x, td, ts = _tab_next(x, agg, Wout[i - 1], wd[i], ws[i])
        gd, gs = _sc_gather(td, ts, dst_g, src_g)
        bias = jnp.stack([bg[i], bc[i]])
        msg2 = _edge_stage(bd2, gd, gs, call[i], bias)
        agg = _sc_scatter(msg2, dst_s).reshape(NC, NPAD, D // 2)

    out = _readout(x, agg, Wout[NCONV - 1],
                   atom_owners.astype(jnp.int32)[:, None],
                   W1, b1[None, :], W2, b2[None, :], W3, b3[None, :])
    return out[0] / jnp.maximum(out[1], 1.0)
